# Initial kernel scaffold; baseline (speedup 1.0000x reference)
import functools

import jax
import jax.numpy as jnp
from jax import lax
from jax.experimental import pallas as pl
from jax.experimental.pallas import tpu as pltpu
from jax.experimental.pallas import tpu_sc as plsc

N_ATOMS = 100000
N_SPECIES = 16
LANES = 16
NUM_WORKERS = 32
CHUNK = 3200
LAST_CHUNK = N_ATOMS - (NUM_WORKERS - 1) * CHUNK


def _body(x_hbm, idx_hbm, scale_hbm, shift_hbm, out_hbm,
          x_v, idx_v, scale_v, shift_v):
    wid = lax.axis_index("s") * 2 + lax.axis_index("c")
    base = wid * CHUNK

    pltpu.sync_copy(scale_hbm, scale_v)
    pltpu.sync_copy(shift_hbm, shift_v)

    def process(n):
        pltpu.sync_copy(x_hbm.at[pl.ds(base, n)], x_v.at[pl.ds(0, n)])
        pltpu.sync_copy(idx_hbm.at[pl.ds(base, n)], idx_v.at[pl.ds(0, n)])

        def step(i, carry):
            off = i * LANES
            idx16 = idx_v[pl.ds(off, LANES)]
            x16 = x_v[pl.ds(off, LANES)]
            s16 = plsc.load_gather(scale_v, [idx16])
            b16 = plsc.load_gather(shift_v, [idx16])
            x_v[pl.ds(off, LANES)] = x16 * s16 + b16
            return carry

        lax.fori_loop(0, n // LANES, step, 0, unroll=4)
        pltpu.sync_copy(x_v.at[pl.ds(0, n)], out_hbm.at[pl.ds(base, n)])

    @pl.when(wid < NUM_WORKERS - 1)
    def _():
        process(CHUNK)

    @pl.when(wid == NUM_WORKERS - 1)
    def _():
        process(LAST_CHUNK)


@jax.jit
def _rescale(x_flat, idx_i32, scale, shift):
    mesh = plsc.VectorSubcoreMesh(core_axis_name="c", subcore_axis_name="s")
    kfn = functools.partial(
        pl.kernel,
        out_type=jax.ShapeDtypeStruct((N_ATOMS,), jnp.float32),
        mesh=mesh,
        scratch_types=[
            pltpu.VMEM((CHUNK,), jnp.float32),
            pltpu.VMEM((CHUNK,), jnp.int32),
            pltpu.VMEM((N_SPECIES,), jnp.float32),
            pltpu.VMEM((N_SPECIES,), jnp.float32),
        ],
    )(_body)
    return kfn(x_flat, idx_i32, scale, shift)


def kernel(x, indices, shift, scale):
    x_flat = x.reshape(-1)
    idx_i32 = indices.astype(jnp.int32)
    out = _rescale(x_flat, idx_i32, scale, shift)
    return out.reshape(-1, 1)


# speedup vs baseline: 1.0353x; 1.0353x over previous
import functools

import jax
import jax.numpy as jnp
from jax import lax
from jax.experimental import pallas as pl
from jax.experimental.pallas import tpu as pltpu
from jax.experimental.pallas import tpu_sc as plsc

N_ATOMS = 100000
N_SPECIES = 16
LANES = 16
NUM_WORKERS = 32
CHUNK = 3200
LAST_CHUNK = N_ATOMS - (NUM_WORKERS - 1) * CHUNK


def _body(x_hbm, idx_hbm, scale_hbm, shift_hbm, out_hbm,
          x_v, idx_v, scale_v, shift_v):
    wid = lax.axis_index("s") * 2 + lax.axis_index("c")
    base = wid * CHUNK

    pltpu.sync_copy(scale_hbm, scale_v)
    pltpu.sync_copy(shift_hbm, shift_v)
    scale_reg = scale_v[...]
    shift_reg = shift_v[...]

    dnums = lax.GatherDimensionNumbers(
        offset_dims=(), collapsed_slice_dims=(0,), start_index_map=(0,))

    def table_lookup(table_reg, idx16):
        return lax.gather(table_reg, idx16[:, None], dnums, slice_sizes=(1,),
                          mode=lax.GatherScatterMode.PROMISE_IN_BOUNDS)

    def process(n):
        pltpu.sync_copy(x_hbm.at[pl.ds(base, n)], x_v.at[pl.ds(0, n)])
        pltpu.sync_copy(idx_hbm.at[pl.ds(base, n)], idx_v.at[pl.ds(0, n)])

        def step(i, carry):
            off = i * LANES
            idx16 = idx_v[pl.ds(off, LANES)]
            x16 = x_v[pl.ds(off, LANES)]
            s16 = table_lookup(scale_reg, idx16)
            b16 = table_lookup(shift_reg, idx16)
            x_v[pl.ds(off, LANES)] = x16 * s16 + b16
            return carry

        lax.fori_loop(0, n // LANES, step, 0, unroll=4)
        pltpu.sync_copy(x_v.at[pl.ds(0, n)], out_hbm.at[pl.ds(base, n)])

    @pl.when(wid < NUM_WORKERS - 1)
    def _():
        process(CHUNK)

    @pl.when(wid == NUM_WORKERS - 1)
    def _():
        process(LAST_CHUNK)


@jax.jit
def _rescale(x_flat, idx_i32, scale, shift):
    mesh = plsc.VectorSubcoreMesh(core_axis_name="c", subcore_axis_name="s")
    kfn = functools.partial(
        pl.kernel,
        out_type=jax.ShapeDtypeStruct((N_ATOMS,), jnp.float32),
        mesh=mesh,
        scratch_types=[
            pltpu.VMEM((CHUNK,), jnp.float32),
            pltpu.VMEM((CHUNK,), jnp.int32),
            pltpu.VMEM((N_SPECIES,), jnp.float32),
            pltpu.VMEM((N_SPECIES,), jnp.float32),
        ],
    )(_body)
    return kfn(x_flat, idx_i32, scale, shift)


def kernel(x, indices, shift, scale):
    x_flat = x.reshape(-1)
    idx_i32 = indices.astype(jnp.int32)
    out = _rescale(x_flat, idx_i32, scale, shift)
    return out.reshape(-1, 1)


# speedup vs baseline: 1.1044x; 1.0667x over previous
import functools

import jax
import jax.numpy as jnp
from jax import lax
from jax.experimental import pallas as pl
from jax.experimental.pallas import tpu as pltpu
from jax.experimental.pallas import tpu_sc as plsc

N_ATOMS = 100000
N_SPECIES = 16
LANES = 16
NUM_WORKERS = 32
CHUNK = 3200
LAST_CHUNK = N_ATOMS - (NUM_WORKERS - 1) * CHUNK


def _body(idx_hbm, scale_hbm, shift_hbm, s_hbm, b_hbm,
          idx_v, s_v, b_v, scale_v, shift_v):
    wid = lax.axis_index("s") * 2 + lax.axis_index("c")
    base = wid * CHUNK

    pltpu.sync_copy(scale_hbm, scale_v)
    pltpu.sync_copy(shift_hbm, shift_v)
    scale_reg = scale_v[...]
    shift_reg = shift_v[...]

    dnums = lax.GatherDimensionNumbers(
        offset_dims=(), collapsed_slice_dims=(0,), start_index_map=(0,))

    def table_lookup(table_reg, idx16):
        return lax.gather(table_reg, idx16[:, None], dnums, slice_sizes=(1,),
                          mode=lax.GatherScatterMode.PROMISE_IN_BOUNDS)

    def process(n):
        pltpu.sync_copy(idx_hbm.at[pl.ds(base, n)], idx_v.at[pl.ds(0, n)])

        @plsc.parallel_loop(0, n // LANES, unroll=8)
        def step(i):
            off = i * LANES
            idx16 = idx_v[pl.ds(off, LANES)]
            s_v[pl.ds(off, LANES)] = table_lookup(scale_reg, idx16)
            b_v[pl.ds(off, LANES)] = table_lookup(shift_reg, idx16)

        pltpu.sync_copy(s_v.at[pl.ds(0, n)], s_hbm.at[pl.ds(base, n)])
        pltpu.sync_copy(b_v.at[pl.ds(0, n)], b_hbm.at[pl.ds(base, n)])

    @pl.when(wid < NUM_WORKERS - 1)
    def _():
        process(CHUNK)

    @pl.when(wid == NUM_WORKERS - 1)
    def _():
        process(LAST_CHUNK)


@jax.jit
def _rescale(x_2d, idx_i32, scale, shift):
    mesh = plsc.VectorSubcoreMesh(core_axis_name="c", subcore_axis_name="s")
    kfn = functools.partial(
        pl.kernel,
        out_type=(jax.ShapeDtypeStruct((N_ATOMS,), jnp.float32),
                  jax.ShapeDtypeStruct((N_ATOMS,), jnp.float32)),
        mesh=mesh,
        scratch_types=[
            pltpu.VMEM((CHUNK,), jnp.int32),
            pltpu.VMEM((CHUNK,), jnp.float32),
            pltpu.VMEM((CHUNK,), jnp.float32),
            pltpu.VMEM((N_SPECIES,), jnp.float32),
            pltpu.VMEM((N_SPECIES,), jnp.float32),
        ],
    )(_body)
    s_arr, b_arr = kfn(idx_i32, scale, shift)
    return x_2d * s_arr[:, None] + b_arr[:, None]


def kernel(x, indices, shift, scale):
    idx_i32 = indices.astype(jnp.int32)
    return _rescale(x, idx_i32, scale, shift)


# speedup vs baseline: 1.1141x; 1.0088x over previous
import functools

import jax
import jax.numpy as jnp
from jax import lax
from jax.experimental import pallas as pl
from jax.experimental.pallas import tpu as pltpu
from jax.experimental.pallas import tpu_sc as plsc

N_ATOMS = 100000
N_SPECIES = 16
LANES = 16
NUM_WORKERS = 32
CHUNK = 3200
LAST_CHUNK = N_ATOMS - (NUM_WORKERS - 1) * CHUNK


def _body(idx_hbm, scale_hbm, shift_hbm, s_hbm, b_hbm,
          idx_v, s_v, b_v, scale_v, shift_v):
    wid = lax.axis_index("s") * 2 + lax.axis_index("c")
    base = lax.min(wid * CHUNK, N_ATOMS - CHUNK)

    pltpu.sync_copy(scale_hbm, scale_v)
    pltpu.sync_copy(shift_hbm, shift_v)
    scale_reg = scale_v[...]
    shift_reg = shift_v[...]

    dnums = lax.GatherDimensionNumbers(
        offset_dims=(), collapsed_slice_dims=(0,), start_index_map=(0,))

    def table_lookup(table_reg, idx16):
        return lax.gather(table_reg, idx16[:, None], dnums, slice_sizes=(1,),
                          mode=lax.GatherScatterMode.PROMISE_IN_BOUNDS)

    pltpu.sync_copy(idx_hbm.at[pl.ds(base, CHUNK)], idx_v)

    @plsc.parallel_loop(0, CHUNK // LANES, unroll=4)
    def step(i):
        off = i * LANES
        idx16 = idx_v[pl.ds(off, LANES)]
        s_v[pl.ds(off, LANES)] = table_lookup(scale_reg, idx16)
        b_v[pl.ds(off, LANES)] = table_lookup(shift_reg, idx16)

    pltpu.sync_copy(s_v, s_hbm.at[pl.ds(base, CHUNK)])
    pltpu.sync_copy(b_v, b_hbm.at[pl.ds(base, CHUNK)])


@jax.jit
def _rescale(x_2d, idx_i32, scale, shift):
    mesh = plsc.VectorSubcoreMesh(core_axis_name="c", subcore_axis_name="s")
    kfn = functools.partial(
        pl.kernel,
        out_type=(jax.ShapeDtypeStruct((N_ATOMS,), jnp.float32),
                  jax.ShapeDtypeStruct((N_ATOMS,), jnp.float32)),
        mesh=mesh,
        scratch_types=[
            pltpu.VMEM((CHUNK,), jnp.int32),
            pltpu.VMEM((CHUNK,), jnp.float32),
            pltpu.VMEM((CHUNK,), jnp.float32),
            pltpu.VMEM((N_SPECIES,), jnp.float32),
            pltpu.VMEM((N_SPECIES,), jnp.float32),
        ],
    )(_body)
    s_arr, b_arr = kfn(idx_i32, scale, shift)
    return x_2d * s_arr[:, None] + b_arr[:, None]


def kernel(x, indices, shift, scale):
    idx_i32 = indices.astype(jnp.int32)
    return _rescale(x, idx_i32, scale, shift)
